# Initial kernel scaffold; baseline (speedup 1.0000x reference)
import jax
import jax.numpy as jnp
from jax.experimental import pallas as pl


def kernel(x, vote_6d, scale, vote_presence_logit, batch):
    raise NotImplementedError("write your pallas kernel here")


# speedup vs baseline: 26.1236x; 26.1236x over previous
import functools
import math

import jax
import jax.numpy as jnp
from jax.experimental import pallas as pl
from jax.experimental.pallas import tpu as pltpu

N = 32768
B = 16
NC = 16
NV = 8
EPS = 1e-10
NVOTE = NC * NV
NFEAT = 8
LOG2PI = math.log(2.0 * math.pi)

ROWS = 512
GRID = N // ROWS


def _body(x_ref, mu_ref, scale_ref, logit_ref, batch_ref,
          lpe_ref, mean_ref, w_ref):
    i = pl.program_id(0)

    @pl.when(i == 0)
    def _init_params():
        s = jnp.maximum(scale_ref[...], EPS)
        inv_s2 = 1.0 / (s * s)
        mu = mu_ref[...]
        musq = jnp.zeros((B, NVOTE), jnp.float32)
        for d in range(6):
            md = mu[16 * d:16 * (d + 1), :]
            musq = musq + md * md
            w_ref[16 * d:16 * (d + 1), :] = md * inv_s2
        w_ref[96:112, :] = -0.5 * inv_s2
        w_ref[112:128, :] = (logit_ref[...] - 6.0 * jnp.log(s)
                             - 3.0 * LOG2PI - 0.5 * inv_s2 * musq)
        lpe_ref[...] = jnp.zeros((1, B), jnp.float32)

    xblk = x_ref[...]
    b = batch_ref[...]
    lane = jax.lax.broadcasted_iota(jnp.int32, (ROWS, NVOTE), 1)
    d_idx = lane >> 4
    b_idx = lane & 15
    norm2 = jnp.sum(xblk * xblk, axis=1, keepdims=True)
    feat = jnp.where(d_idx == 6, norm2, 1.0)
    for d in range(6):
        feat = jnp.where(d_idx == d, xblk[:, d:d + 1], feat)
    xb = jnp.where(b_idx == b, feat, 0.0)

    post = jax.lax.dot_general(
        xb, w_ref[...], (((1,), (0,)), ((), ())),
        preferred_element_type=jnp.float32,
        precision=jax.lax.Precision.HIGHEST)

    m = jnp.max(post, axis=1, keepdims=True)
    ssum = jnp.sum(jnp.exp(post - m), axis=1, keepdims=True)
    lpp = m + jnp.log(ssum)

    b16 = jax.lax.broadcasted_iota(jnp.int32, (ROWS, B), 1)
    contrib = jnp.where(b16 == b, lpp, 0.0)
    lpe_ref[...] += jnp.sum(contrib, axis=0, keepdims=True)
    mean_ref[...] = jnp.sum(lpe_ref[...], axis=1, keepdims=True) / B


@jax.jit
def kernel(x, vote_6d, scale, vote_presence_logit, batch):
    mu_t = jnp.transpose(vote_6d, (3, 0, 1, 2)).reshape(6 * B, NVOTE)
    scale_r = scale.reshape(B, NVOTE)
    logit_r = vote_presence_logit.reshape(B, NVOTE)
    batch_2d = batch.reshape(N, 1)

    lpe, mean = pl.pallas_call(
        _body,
        grid=(GRID,),
        in_specs=[
            pl.BlockSpec((ROWS, 6), lambda i: (i, 0)),
            pl.BlockSpec((6 * B, NVOTE), lambda i: (0, 0)),
            pl.BlockSpec((B, NVOTE), lambda i: (0, 0)),
            pl.BlockSpec((B, NVOTE), lambda i: (0, 0)),
            pl.BlockSpec((ROWS, 1), lambda i: (i, 0)),
        ],
        out_specs=[
            pl.BlockSpec((1, B), lambda i: (0, 0)),
            pl.BlockSpec((1, 1), lambda i: (0, 0)),
        ],
        out_shape=[
            jax.ShapeDtypeStruct((1, B), jnp.float32),
            jax.ShapeDtypeStruct((1, 1), jnp.float32),
        ],
        scratch_shapes=[pltpu.VMEM((NVOTE, NVOTE), jnp.float32)],
    )(x, mu_t, scale_r, logit_r, batch_2d)
    return (mean.reshape(()), lpe.reshape(B))


# speedup vs baseline: 31.3215x; 1.1990x over previous
import functools
import math

import jax
import jax.numpy as jnp
from jax.experimental import pallas as pl
from jax.experimental.pallas import tpu as pltpu

N = 32768
B = 16
NC = 16
NV = 8
EPS = 1e-10
NVOTE = NC * NV
NFEAT = 8
LOG2PI = math.log(2.0 * math.pi)

ROWS = 1024
GRID = N // ROWS


def _body(x_ref, mu_ref, scale_ref, logit_ref, batch_ref,
          lpe_ref, mean_ref, w_ref):
    i = pl.program_id(0)

    @pl.when(i == 0)
    def _init_params():
        s = jnp.maximum(scale_ref[...], EPS)
        inv_s2 = 1.0 / (s * s)
        mu = mu_ref[...]
        musq = jnp.zeros((B, NVOTE), jnp.float32)
        for d in range(6):
            md = mu[16 * d:16 * (d + 1), :]
            musq = musq + md * md
            w_ref[16 * d:16 * (d + 1), :] = md * inv_s2
        w_ref[96:112, :] = -0.5 * inv_s2
        w_ref[112:128, :] = (logit_ref[...] - 6.0 * jnp.log(s)
                             - 3.0 * LOG2PI - 0.5 * inv_s2 * musq)
        lpe_ref[...] = jnp.zeros((1, B), jnp.float32)

    xblk = x_ref[...]
    b = batch_ref[...]
    lane = jax.lax.broadcasted_iota(jnp.int32, (ROWS, NVOTE), 1)
    d_idx = lane >> 4
    b_idx = lane & 15
    norm2 = jnp.sum(xblk * xblk, axis=1, keepdims=True)
    feat = jnp.where(d_idx == 6, norm2, 1.0)
    for d in range(6):
        feat = jnp.where(d_idx == d, xblk[:, d:d + 1], feat)
    xb = jnp.where(b_idx == b, feat, 0.0)

    post = jax.lax.dot_general(
        xb, w_ref[...], (((1,), (0,)), ((), ())),
        preferred_element_type=jnp.float32,
        precision=jax.lax.Precision.HIGHEST)

    m = jnp.max(post, axis=1, keepdims=True)
    ssum = jnp.sum(jnp.exp(post - m), axis=1, keepdims=True)
    lpp = m + jnp.log(ssum)

    b16 = jax.lax.broadcasted_iota(jnp.int32, (ROWS, B), 1)
    contrib = jnp.where(b16 == b, lpp, 0.0)
    lpe_ref[...] += jnp.sum(contrib, axis=0, keepdims=True)
    mean_ref[...] = jnp.sum(lpe_ref[...], axis=1, keepdims=True) / B


@jax.jit
def kernel(x, vote_6d, scale, vote_presence_logit, batch):
    mu_t = jnp.transpose(vote_6d, (3, 0, 1, 2)).reshape(6 * B, NVOTE)
    scale_r = scale.reshape(B, NVOTE)
    logit_r = vote_presence_logit.reshape(B, NVOTE)
    batch_2d = batch.reshape(N, 1)

    lpe, mean = pl.pallas_call(
        _body,
        grid=(GRID,),
        in_specs=[
            pl.BlockSpec((ROWS, 6), lambda i: (i, 0)),
            pl.BlockSpec((6 * B, NVOTE), lambda i: (0, 0)),
            pl.BlockSpec((B, NVOTE), lambda i: (0, 0)),
            pl.BlockSpec((B, NVOTE), lambda i: (0, 0)),
            pl.BlockSpec((ROWS, 1), lambda i: (i, 0)),
        ],
        out_specs=[
            pl.BlockSpec((1, B), lambda i: (0, 0)),
            pl.BlockSpec((1, 1), lambda i: (0, 0)),
        ],
        out_shape=[
            jax.ShapeDtypeStruct((1, B), jnp.float32),
            jax.ShapeDtypeStruct((1, 1), jnp.float32),
        ],
        scratch_shapes=[pltpu.VMEM((NVOTE, NVOTE), jnp.float32)],
    )(x, mu_t, scale_r, logit_r, batch_2d)
    return (mean.reshape(()), lpe.reshape(B))
